# Initial kernel scaffold; baseline (speedup 1.0000x reference)
import jax
import jax.numpy as jnp
from jax.experimental import pallas as pl


def kernel(note_tokens, note_durs, note_types, emb_weight, type_emb_weight, dur_w, dur_b):
    raise NotImplementedError("write your pallas kernel here")


# speedup vs baseline: 4.0342x; 4.0342x over previous
import jax
import jax.numpy as jnp
from jax import lax
from jax.experimental import pallas as pl
from jax.experimental.pallas import tpu as pltpu
from jax.experimental.pallas import tpu_sc as plsc

D = 64
B, L = 4096, 200
BF = B * L
SCALE = 8.0
NC, NS = 2, 16
NW = NC * NS
PER_W = BF // NW
CH = 1024
NCH = PER_W // CH
KS = CH // 128


def _sc_body(tok_hbm, typ_hbm, dur_hbm, emb_hbm, temb_hbm, dw_hbm, db_hbm,
             out_hbm,
             idx_v, typ_v, dur_v, rows_v, temb_v, comb_v, dw_v, db_v, gsem):
    wid = lax.axis_index("s") * NC + lax.axis_index("c")

    pltpu.sync_copy(temb_hbm, temb_v)
    pltpu.sync_copy(dw_hbm, dw_v)
    pltpu.sync_copy(db_hbm, db_v)
    for t in range(5):
        for c in range(0, D, 16):
            comb_v[t, pl.ds(c, 16)] = (
                temb_v[t, pl.ds(c, 16)] * SCALE + db_v[pl.ds(c, 16)]
            )

    iota = lax.iota(jnp.int32, 16)

    def chunk_body(g, carry):
        base = wid * PER_W + g * CH
        pltpu.sync_copy(tok_hbm.at[pl.ds(pl.multiple_of(base // 128, 8), KS)],
                        idx_v)
        pltpu.sync_copy(typ_hbm.at[pl.ds(base, CH)], typ_v)
        pltpu.sync_copy(dur_hbm.at[pl.ds(base, CH)], dur_v)
        descs = [
            pltpu.async_copy(emb_hbm.at[idx_v.at[j]],
                             rows_v.at[pl.ds(j * 128, 128)], gsem)
            for j in range(KS)
        ]
        for d in descs:
            d.wait()

        def row_body(i16, c2):
            i0 = i16 * 16
            typ16 = typ_v[pl.ds(i0, 16)]
            dur16 = dur_v[pl.ds(i0, 16)]
            for k in range(16):
                typ = typ16[k]
                durv = jnp.full((16,), dur16[k], jnp.float32)
                for c in range(0, D, 16):
                    emb = rows_v[i0 + k, pl.ds(c, 16)]
                    cmb = comb_v[typ, pl.ds(c, 16)]
                    dwc = dw_v[pl.ds(c, 16)]
                    rows_v[i0 + k, pl.ds(c, 16)] = emb * SCALE + cmb + durv * dwc
            return c2

        lax.fori_loop(0, CH // 16, row_body, 0)
        pltpu.sync_copy(rows_v, out_hbm.at[pl.ds(base, CH)])
        return carry

    lax.fori_loop(0, NCH, chunk_body, 0)


def kernel(note_tokens, note_durs, note_types, emb_weight, type_emb_weight,
           dur_w, dur_b):
    tok2 = note_tokens.reshape(BF // 128, 128)
    typf = note_types.reshape(BF)
    durf = note_durs.reshape(BF)
    mesh = plsc.VectorSubcoreMesh(core_axis_name="c", subcore_axis_name="s")
    out = pl.kernel(
        _sc_body,
        out_type=jax.ShapeDtypeStruct((BF, D), jnp.float32),
        mesh=mesh,
        compiler_params=pltpu.CompilerParams(use_tc_tiling_on_sc=False),
        scratch_types=[
            pltpu.VMEM((KS, 128), jnp.int32),
            pltpu.VMEM((CH,), jnp.int32),
            pltpu.VMEM((CH,), jnp.float32),
            pltpu.VMEM((CH, D), jnp.float32),
            pltpu.VMEM((5, D), jnp.float32),
            pltpu.VMEM((5, D), jnp.float32),
            pltpu.VMEM((D,), jnp.float32),
            pltpu.VMEM((D,), jnp.float32),
            pltpu.SemaphoreType.DMA,
        ],
    )(tok2, typf, durf, emb_weight, type_emb_weight, dur_w, dur_b)
    return out.reshape(B, L, D)


# speedup vs baseline: 4.4314x; 1.0985x over previous
import jax
import jax.numpy as jnp
from jax import lax
from jax.experimental import pallas as pl
from jax.experimental.pallas import tpu as pltpu
from jax.experimental.pallas import tpu_sc as plsc

D = 64
B, L = 4096, 200
BF = B * L
SCALE = 8.0
NC, NS = 2, 16
NW = NC * NS
PER_W = BF // NW
CH = 512
NCHUNK = PER_W // CH
KSUB = CH // 128


def _sc_body(tok_hbm, typ_hbm, dur_hbm, emb_hbm, temb_hbm, dw_hbm, db_hbm,
             out_hbm,
             idx_v, typ_v, dur_v, rows_v, temb_v, comb_v, dw_v, db_v,
             isem, gsem, ssem, wsem):
    wid = lax.axis_index("s") * NC + lax.axis_index("c")
    last = NCHUNK - 1

    def start(k):
        return wid * PER_W + k * CH

    def idx_copy(k, par):
        return pltpu.make_async_copy(
            tok_hbm.at[pl.ds(start(k), CH)], idx_v.at[par], isem)

    def small_copies(k, par):
        return (
            pltpu.make_async_copy(
                typ_hbm.at[pl.ds(start(k), CH)], typ_v.at[par], ssem),
            pltpu.make_async_copy(
                dur_hbm.at[pl.ds(start(k), CH)], dur_v.at[par], ssem),
        )

    def gather_copies(par):
        return [
            pltpu.make_async_copy(
                emb_hbm.at[idx_v.at[par, pl.ds(j * 128, 128)]],
                rows_v.at[par, pl.ds(j * 128, 128)], gsem)
            for j in range(KSUB)
        ]

    def out_copy(k, par):
        return pltpu.make_async_copy(
            rows_v.at[par], out_hbm.at[pl.ds(start(k), CH)], wsem)

    pltpu.sync_copy(temb_hbm, temb_v)
    pltpu.sync_copy(dw_hbm, dw_v)
    pltpu.sync_copy(db_hbm, db_v)
    for t in range(5):
        for c in range(0, D, 16):
            comb_v[t, pl.ds(c, 16)] = (
                temb_v[t, pl.ds(c, 16)] * SCALE + db_v[pl.ds(c, 16)]
            )

    def compute(par):
        def row_body(i16, c2):
            i0 = i16 * 16
            typ16 = typ_v[par, pl.ds(i0, 16)]
            dur16 = dur_v[par, pl.ds(i0, 16)]
            for k in range(16):
                typ = typ16[k]
                durv = jnp.full((16,), dur16[k], jnp.float32)
                for c in range(0, D, 16):
                    emb = rows_v[par, i0 + k, pl.ds(c, 16)]
                    cmb = comb_v[typ, pl.ds(c, 16)]
                    dwc = dw_v[pl.ds(c, 16)]
                    rows_v[par, i0 + k, pl.ds(c, 16)] = (
                        emb * SCALE + cmb + durv * dwc)
            return c2

        lax.fori_loop(0, CH // 16, row_body, 0)

    def process(k, par):
        for d in gather_copies(par):
            d.wait()
        for d in small_copies(k, par):
            d.wait()

        @pl.when(k > 0)
        def _():
            out_copy(k - 1, 1 - par).wait()

        idx_copy(jnp.minimum(k + 1, last), 1 - par).wait()
        idx_copy(jnp.minimum(k + 2, last), par).start()
        nxt = jnp.minimum(k + 1, last)
        for d in gather_copies(1 - par):
            d.start()
        for d in small_copies(nxt, 1 - par):
            d.start()
        compute(par)
        out_copy(k, par).start()

    idx_copy(0, 0).start()
    idx_copy(0, 0).wait()
    idx_copy(1, 1).start()
    for d in gather_copies(0):
        d.start()
    for d in small_copies(0, 0):
        d.start()

    def pair_body(m, carry):
        process(2 * m, 0)
        process(2 * m + 1, 1)
        return carry

    lax.fori_loop(0, NCHUNK // 2, pair_body, 0)

    out_copy(last, 1).wait()
    for d in gather_copies(0):
        d.wait()
    for d in small_copies(last, 0):
        d.wait()
    idx_copy(last, 1).wait()


def kernel(note_tokens, note_durs, note_types, emb_weight, type_emb_weight,
           dur_w, dur_b):
    tokf = note_tokens.reshape(BF)
    typf = note_types.reshape(BF)
    durf = note_durs.reshape(BF)
    mesh = plsc.VectorSubcoreMesh(core_axis_name="c", subcore_axis_name="s")
    out = pl.kernel(
        _sc_body,
        out_type=jax.ShapeDtypeStruct((BF, D), jnp.float32),
        mesh=mesh,
        compiler_params=pltpu.CompilerParams(use_tc_tiling_on_sc=False),
        scratch_types=[
            pltpu.VMEM((2, CH), jnp.int32),
            pltpu.VMEM((2, CH), jnp.int32),
            pltpu.VMEM((2, CH), jnp.float32),
            pltpu.VMEM((2, CH, D), jnp.float32),
            pltpu.VMEM((5, D), jnp.float32),
            pltpu.VMEM((5, D), jnp.float32),
            pltpu.VMEM((D,), jnp.float32),
            pltpu.VMEM((D,), jnp.float32),
            pltpu.SemaphoreType.DMA,
            pltpu.SemaphoreType.DMA,
            pltpu.SemaphoreType.DMA,
            pltpu.SemaphoreType.DMA,
        ],
    )(tokf, typf, durf, emb_weight, type_emb_weight, dur_w, dur_b)
    return out.reshape(B, L, D)


# speedup vs baseline: 6.6099x; 1.4916x over previous
import jax
import jax.numpy as jnp
from jax import lax
from jax.experimental import pallas as pl
from jax.experimental.pallas import tpu as pltpu
from jax.experimental.pallas import tpu_sc as plsc

D = 64
B, L = 4096, 200
BF = B * L
SCALE = 8.0
NC, NS = 2, 16
NW = NC * NS
PER_W = BF // NW
CH = 512
NCHUNK = PER_W // CH
KSUB = CH // 128


def _sc_body(tok_hbm, typ_hbm, dur_hbm, emb_hbm, temb_hbm, dw_hbm, db_hbm,
             out_hbm,
             idx_v, typ_v, dur_v, rows_v, temb_v, comb_v, dw_v, db_v,
             isem, gsem, ssem, wsem):
    wid = lax.axis_index("s") * NC + lax.axis_index("c")
    last = NCHUNK - 1

    def start(k):
        return wid * PER_W + k * CH

    def idx_copy(k, par):
        return pltpu.make_async_copy(
            tok_hbm.at[pl.ds(start(k), CH)], idx_v.at[par], isem)

    def small_copies(k, par):
        return (
            pltpu.make_async_copy(
                typ_hbm.at[pl.ds(start(k), CH)], typ_v.at[par], ssem),
            pltpu.make_async_copy(
                dur_hbm.at[pl.ds(start(k), CH)], dur_v.at[par], ssem),
        )

    def gather_copies(par):
        return [
            pltpu.make_async_copy(
                emb_hbm.at[idx_v.at[par, pl.ds(j * 128, 128)]],
                rows_v.at[par, pl.ds(j * 128, 128)], gsem)
            for j in range(KSUB)
        ]

    def out_copy(k, par):
        return pltpu.make_async_copy(
            rows_v.at[par], out_hbm.at[pl.ds(start(k), CH)], wsem)

    pltpu.sync_copy(temb_hbm, temb_v)
    pltpu.sync_copy(dw_hbm, dw_v)
    pltpu.sync_copy(db_hbm, db_v)
    for t in range(5):
        for c in range(0, D, 16):
            comb_v[t, pl.ds(c, 16)] = (
                temb_v[t, pl.ds(c, 16)] * SCALE + db_v[pl.ds(c, 16)]
            )

    def compute(par):
        dws = tuple(dw_v[pl.ds(c, 16)] for c in range(0, D, 16))

        @plsc.parallel_loop(0, CH // 16, unroll=2, carry=dws)
        def row_body(i16, dwr):
            i0 = i16 * 16
            typ16 = typ_v[par, pl.ds(i0, 16)]
            dur16 = dur_v[par, pl.ds(i0, 16)]
            for k in range(16):
                typ = typ16[k]
                durv = jnp.full((16,), dur16[k], jnp.float32)
                for c in range(0, D, 16):
                    emb = rows_v[par, i0 + k, pl.ds(c, 16)]
                    cmb = comb_v[typ, pl.ds(c, 16)]
                    rows_v[par, i0 + k, pl.ds(c, 16)] = (
                        emb * SCALE + cmb + durv * dwr[c // 16])
            return dwr

    def process(k, par):
        for d in gather_copies(par):
            d.wait()
        for d in small_copies(k, par):
            d.wait()

        @pl.when(k > 0)
        def _():
            out_copy(k - 1, 1 - par).wait()

        idx_copy(jnp.minimum(k + 1, last), 1 - par).wait()
        idx_copy(jnp.minimum(k + 2, last), par).start()
        nxt = jnp.minimum(k + 1, last)
        for d in gather_copies(1 - par):
            d.start()
        for d in small_copies(nxt, 1 - par):
            d.start()
        compute(par)
        out_copy(k, par).start()

    idx_copy(0, 0).start()
    idx_copy(0, 0).wait()
    idx_copy(1, 1).start()
    for d in gather_copies(0):
        d.start()
    for d in small_copies(0, 0):
        d.start()

    def pair_body(m, carry):
        process(2 * m, 0)
        process(2 * m + 1, 1)
        return carry

    lax.fori_loop(0, NCHUNK // 2, pair_body, 0)

    out_copy(last, 1).wait()
    for d in gather_copies(0):
        d.wait()
    for d in small_copies(last, 0):
        d.wait()
    idx_copy(last, 1).wait()


def kernel(note_tokens, note_durs, note_types, emb_weight, type_emb_weight,
           dur_w, dur_b):
    tokf = note_tokens.reshape(BF)
    typf = note_types.reshape(BF)
    durf = note_durs.reshape(BF)
    mesh = plsc.VectorSubcoreMesh(core_axis_name="c", subcore_axis_name="s")
    out = pl.kernel(
        _sc_body,
        out_type=jax.ShapeDtypeStruct((BF, D), jnp.float32),
        mesh=mesh,
        compiler_params=pltpu.CompilerParams(use_tc_tiling_on_sc=False),
        scratch_types=[
            pltpu.VMEM((2, CH), jnp.int32),
            pltpu.VMEM((2, CH), jnp.int32),
            pltpu.VMEM((2, CH), jnp.float32),
            pltpu.VMEM((2, CH, D), jnp.float32),
            pltpu.VMEM((5, D), jnp.float32),
            pltpu.VMEM((5, D), jnp.float32),
            pltpu.VMEM((D,), jnp.float32),
            pltpu.VMEM((D,), jnp.float32),
            pltpu.SemaphoreType.DMA,
            pltpu.SemaphoreType.DMA,
            pltpu.SemaphoreType.DMA,
            pltpu.SemaphoreType.DMA,
        ],
    )(tokf, typf, durf, emb_weight, type_emb_weight, dur_w, dur_b)
    return out.reshape(B, L, D)
